# Initial kernel scaffold; baseline (speedup 1.0000x reference)
import jax
import jax.numpy as jnp
from jax.experimental import pallas as pl


def kernel(logits, temperature, top_p, token_lengths, output_lengths, top_k):
    raise NotImplementedError("write your pallas kernel here")


# speedup vs baseline: 34.0303x; 34.0303x over previous
import jax
import jax.numpy as jnp
from jax.experimental import pallas as pl
from jax.experimental.pallas import tpu as pltpu

_V = 100000
_VPAD = 100096
_KCAP = 48
_KOUT = 64
_NEG_INF = float("-inf")


_RB = 8


def _topk_lse_kernel(x_ref, vals_ref, idxs_ref, lse_ref, scratch_ref):
    x = x_ref[...]
    iota = jax.lax.broadcasted_iota(jnp.int32, x.shape, 1)
    m0 = jnp.max(x, axis=1, keepdims=True)
    s = jnp.sum(jnp.exp(x - m0), axis=1, keepdims=True)
    lse_ref[...] = m0 + jnp.log(s)
    scratch_ref[...] = x

    k_iota = jax.lax.broadcasted_iota(jnp.int32, (_RB, _KOUT), 1)

    def body(k, carry):
        vals, idxs = carry
        xc = scratch_ref[...]
        m = jnp.max(xc, axis=1, keepdims=True)
        idx = jnp.min(jnp.where(xc == m, iota, jnp.int32(2**31 - 1)),
                      axis=1, keepdims=True)
        scratch_ref[...] = jnp.where(iota == idx, _NEG_INF, xc)
        vals = jnp.where(k_iota == k, m, vals)
        idxs = jnp.where(k_iota == k, idx, idxs)
        return vals, idxs

    vals0 = jnp.full((_RB, _KOUT), _NEG_INF, jnp.float32)
    idxs0 = jnp.zeros((_RB, _KOUT), jnp.int32)
    vals, idxs = jax.lax.fori_loop(0, _KCAP, body, (vals0, idxs0))
    vals_ref[...] = vals
    idxs_ref[...] = idxs


def _rank_kernel(sid_ref, x_ref, xv_ref, rank_ref):
    x = x_ref[...]
    iota = jax.lax.broadcasted_iota(jnp.int32, x.shape, 1)
    sid = sid_ref[...]
    xv = jnp.max(jnp.where(iota == sid, x, _NEG_INF), axis=1, keepdims=True)
    rank_ref[...] = jnp.sum((x >= xv).astype(jnp.int32), axis=1, keepdims=True)
    xv_ref[...] = xv


def kernel(logits, temperature, top_p, token_lengths, output_lengths, top_k):
    logits = logits.astype(jnp.float32)
    B, V = logits.shape
    xpad = jnp.pad(logits, ((0, 0), (0, _VPAD - V)), constant_values=_NEG_INF)

    vals, idxs, lse2 = pl.pallas_call(
        _topk_lse_kernel,
        grid=(B // _RB,),
        in_specs=[pl.BlockSpec((_RB, _VPAD), lambda i: (i, 0))],
        out_specs=[
            pl.BlockSpec((_RB, _KOUT), lambda i: (i, 0)),
            pl.BlockSpec((_RB, _KOUT), lambda i: (i, 0)),
            pl.BlockSpec((_RB, 1), lambda i: (i, 0)),
        ],
        out_shape=[
            jax.ShapeDtypeStruct((B, _KOUT), jnp.float32),
            jax.ShapeDtypeStruct((B, _KOUT), jnp.int32),
            jax.ShapeDtypeStruct((B, 1), jnp.float32),
        ],
        scratch_shapes=[pltpu.VMEM((_RB, _VPAD), jnp.float32)],
    )(xpad)
    lse = lse2[:, 0]

    v48 = vals[:, :_KCAP]
    i48 = idxs[:, :_KCAP]

    kidx = jnp.clip(jnp.asarray(top_k, jnp.int32) - 1, 0, _KCAP - 1)
    sv = v48 / temperature[:, None]
    thresh = jnp.take_along_axis(
        sv, jnp.broadcast_to(kidx.reshape(1, 1), (B, 1)), axis=-1)
    survivor = sv >= thresh
    q = jnp.where(survivor,
                  jax.nn.softmax(jnp.where(survivor, sv, _NEG_INF), axis=-1),
                  0.0)
    prefix = jnp.cumsum(q, axis=-1) - q
    pos = jnp.arange(_KCAP)[None, :]
    keep = survivor & ((prefix < top_p[:, None]) | (pos == 0))

    s_kept = jnp.sum(jnp.where(keep, jnp.exp(sv - sv[:, :1]), 0.0), axis=-1)
    max_logprobs = -jnp.log(s_kept)

    n_kept = jnp.sum(keep.astype(jnp.int32), axis=-1)
    order = jnp.argsort(jnp.where(keep, pos, _KCAP + pos), axis=-1)
    kept_ids_sorted = jnp.take_along_axis(i48, order[:, :10], axis=-1)

    small = jnp.arange(64)[None, :]
    kept_ids_all = jnp.where(keep, i48, -1)
    is_kept_id = jnp.any(small[:, :, None] == kept_ids_all[:, None, :], axis=-1)
    filler = jnp.sort(jnp.where(is_kept_id, 10000 + small, small), axis=-1)[:, :10]

    r10 = jnp.arange(10)[None, :]
    fill_pos = jnp.clip(r10 - n_kept[:, None], 0, 9)
    filler_pick = jnp.take_along_axis(filler, fill_pos, axis=-1)
    merged_ids = jnp.where(r10 < n_kept[:, None], kept_ids_sorted, filler_pick)

    lengths = token_lengths[jnp.clip(merged_ids, 0, token_lengths.shape[0] - 1)]
    longest = jnp.argmax(lengths, axis=-1)
    final = jnp.take_along_axis(merged_ids, longest[:, None], axis=-1)[:, 0]

    force = (output_lengths >= 64) & (max_logprobs < -0.7)
    sampled = jnp.where(force, 2, final)
    sampled = jnp.where(temperature < 1e-5, i48[:, 0], sampled).astype(jnp.int32)

    xv2, rank2 = pl.pallas_call(
        _rank_kernel,
        grid=(B // _RB,),
        in_specs=[
            pl.BlockSpec((_RB, 1), lambda i: (i, 0)),
            pl.BlockSpec((_RB, _VPAD), lambda i: (i, 0)),
        ],
        out_specs=[
            pl.BlockSpec((_RB, 1), lambda i: (i, 0)),
            pl.BlockSpec((_RB, 1), lambda i: (i, 0)),
        ],
        out_shape=[
            jax.ShapeDtypeStruct((B, 1), jnp.float32),
            jax.ShapeDtypeStruct((B, 1), jnp.int32),
        ],
    )(sampled[:, None], xpad)

    token_logprobs = xv2[:, 0] - lse
    indices = jnp.concatenate([sampled[:, None], i48[:, :20]], axis=1)
    lp = jnp.concatenate(
        [token_logprobs[:, None], v48[:, :20] - lse[:, None]], axis=1)
    return sampled[:, None], indices.astype(jnp.int32), lp, rank2[:, 0]


# speedup vs baseline: 39.3541x; 1.1564x over previous
import jax
import jax.numpy as jnp
from jax.experimental import pallas as pl
from jax.experimental.pallas import tpu as pltpu

_V = 100000
_VPAD = 100096
_KCAP = 48
_KOUT = 64
_NEG_INF = float("-inf")


_RB = 8


def _topk_lse_kernel(x_ref, vals_ref, idxs_ref, lse_ref):
    x = x_ref[...]
    iota = jax.lax.broadcasted_iota(jnp.int32, x.shape, 1)
    m0 = jnp.max(x, axis=1, keepdims=True)
    s = jnp.sum(jnp.exp(x - m0), axis=1, keepdims=True)
    lse_ref[...] = m0 + jnp.log(s)

    k_iota = jax.lax.broadcasted_iota(jnp.int32, (_RB, _KOUT), 1)

    def body(k, carry):
        vals, idxs, m_prev, i_prev = carry
        cand = jnp.where((x < m_prev) | ((x == m_prev) & (iota > i_prev)),
                         x, _NEG_INF)
        m = jnp.max(cand, axis=1, keepdims=True)
        idx = jnp.min(jnp.where(cand == m, iota, jnp.int32(2**31 - 1)),
                      axis=1, keepdims=True)
        vals = jnp.where(k_iota == k, m, vals)
        idxs = jnp.where(k_iota == k, idx, idxs)
        return vals, idxs, m, idx

    vals0 = jnp.full((_RB, _KOUT), _NEG_INF, jnp.float32)
    idxs0 = jnp.zeros((_RB, _KOUT), jnp.int32)
    m0c = jnp.full((_RB, 1), jnp.inf, jnp.float32)
    i0c = jnp.full((_RB, 1), -1, jnp.int32)
    vals, idxs, _, _ = jax.lax.fori_loop(0, _KCAP, body,
                                         (vals0, idxs0, m0c, i0c))
    vals_ref[...] = vals
    idxs_ref[...] = idxs


def _rank_kernel(sid_ref, x_ref, xv_ref, rank_ref):
    x = x_ref[...]
    iota = jax.lax.broadcasted_iota(jnp.int32, x.shape, 1)
    sid = sid_ref[...]
    xv = jnp.max(jnp.where(iota == sid, x, _NEG_INF), axis=1, keepdims=True)
    rank_ref[...] = jnp.sum((x >= xv).astype(jnp.int32), axis=1, keepdims=True)
    xv_ref[...] = xv


def kernel(logits, temperature, top_p, token_lengths, output_lengths, top_k):
    logits = logits.astype(jnp.float32)
    B, V = logits.shape
    xpad = jnp.pad(logits, ((0, 0), (0, _VPAD - V)), constant_values=_NEG_INF)

    vals, idxs, lse2 = pl.pallas_call(
        _topk_lse_kernel,
        grid=(B // _RB,),
        in_specs=[pl.BlockSpec((_RB, _VPAD), lambda i: (i, 0))],
        out_specs=[
            pl.BlockSpec((_RB, _KOUT), lambda i: (i, 0)),
            pl.BlockSpec((_RB, _KOUT), lambda i: (i, 0)),
            pl.BlockSpec((_RB, 1), lambda i: (i, 0)),
        ],
        out_shape=[
            jax.ShapeDtypeStruct((B, _KOUT), jnp.float32),
            jax.ShapeDtypeStruct((B, _KOUT), jnp.int32),
            jax.ShapeDtypeStruct((B, 1), jnp.float32),
        ],
    )(xpad)
    lse = lse2[:, 0]

    v48 = vals[:, :_KCAP]
    i48 = idxs[:, :_KCAP]

    kidx = jnp.clip(jnp.asarray(top_k, jnp.int32) - 1, 0, _KCAP - 1)
    sv = v48 / temperature[:, None]
    thresh = jnp.take_along_axis(
        sv, jnp.broadcast_to(kidx.reshape(1, 1), (B, 1)), axis=-1)
    survivor = sv >= thresh
    q = jnp.where(survivor,
                  jax.nn.softmax(jnp.where(survivor, sv, _NEG_INF), axis=-1),
                  0.0)
    prefix = jnp.cumsum(q, axis=-1) - q
    pos = jnp.arange(_KCAP)[None, :]
    keep = survivor & ((prefix < top_p[:, None]) | (pos == 0))

    s_kept = jnp.sum(jnp.where(keep, jnp.exp(sv - sv[:, :1]), 0.0), axis=-1)
    max_logprobs = -jnp.log(s_kept)

    n_kept = jnp.sum(keep.astype(jnp.int32), axis=-1)
    order = jnp.argsort(jnp.where(keep, pos, _KCAP + pos), axis=-1)
    kept_ids_sorted = jnp.take_along_axis(i48, order[:, :10], axis=-1)

    small = jnp.arange(64)[None, :]
    kept_ids_all = jnp.where(keep, i48, -1)
    is_kept_id = jnp.any(small[:, :, None] == kept_ids_all[:, None, :], axis=-1)
    filler = jnp.sort(jnp.where(is_kept_id, 10000 + small, small), axis=-1)[:, :10]

    r10 = jnp.arange(10)[None, :]
    fill_pos = jnp.clip(r10 - n_kept[:, None], 0, 9)
    filler_pick = jnp.take_along_axis(filler, fill_pos, axis=-1)
    merged_ids = jnp.where(r10 < n_kept[:, None], kept_ids_sorted, filler_pick)

    lengths = token_lengths[jnp.clip(merged_ids, 0, token_lengths.shape[0] - 1)]
    longest = jnp.argmax(lengths, axis=-1)
    final = jnp.take_along_axis(merged_ids, longest[:, None], axis=-1)[:, 0]

    force = (output_lengths >= 64) & (max_logprobs < -0.7)
    sampled = jnp.where(force, 2, final)
    sampled = jnp.where(temperature < 1e-5, i48[:, 0], sampled).astype(jnp.int32)

    xv2, rank2 = pl.pallas_call(
        _rank_kernel,
        grid=(B // _RB,),
        in_specs=[
            pl.BlockSpec((_RB, 1), lambda i: (i, 0)),
            pl.BlockSpec((_RB, _VPAD), lambda i: (i, 0)),
        ],
        out_specs=[
            pl.BlockSpec((_RB, 1), lambda i: (i, 0)),
            pl.BlockSpec((_RB, 1), lambda i: (i, 0)),
        ],
        out_shape=[
            jax.ShapeDtypeStruct((B, 1), jnp.float32),
            jax.ShapeDtypeStruct((B, 1), jnp.int32),
        ],
    )(sampled[:, None], xpad)

    token_logprobs = xv2[:, 0] - lse
    indices = jnp.concatenate([sampled[:, None], i48[:, :20]], axis=1)
    lp = jnp.concatenate(
        [token_logprobs[:, None], v48[:, :20] - lse[:, None]], axis=1)
    return sampled[:, None], indices.astype(jnp.int32), lp, rank2[:, 0]


# speedup vs baseline: 48.7699x; 1.2393x over previous
import jax
import jax.numpy as jnp
from jax.experimental import pallas as pl
from jax.experimental.pallas import tpu as pltpu

_V = 100000
_VPAD = 100096
_KCAP = 48
_KOUT = 64
_NEG_INF = float("-inf")


_RB = 16


def _topk_lse_kernel(x_ref, vals_ref, idxs_ref, lse_ref):
    x = x_ref[...]
    iota = jax.lax.broadcasted_iota(jnp.int32, x.shape, 1)
    m0 = jnp.max(x, axis=1, keepdims=True)
    s = jnp.sum(jnp.exp(x - m0), axis=1, keepdims=True)
    lse_ref[...] = m0 + jnp.log(s)

    k_iota = jax.lax.broadcasted_iota(jnp.int32, (_RB, _KOUT), 1)

    def body(k, carry):
        vals, idxs, m_prev, i_prev = carry
        cand = jnp.where((x < m_prev) | ((x == m_prev) & (iota > i_prev)),
                         x, _NEG_INF)
        m = jnp.max(cand, axis=1, keepdims=True)
        idx = jnp.min(jnp.where(cand == m, iota, jnp.int32(2**31 - 1)),
                      axis=1, keepdims=True)
        vals = jnp.where(k_iota == k, m, vals)
        idxs = jnp.where(k_iota == k, idx, idxs)
        return vals, idxs, m, idx

    vals0 = jnp.full((_RB, _KOUT), _NEG_INF, jnp.float32)
    idxs0 = jnp.zeros((_RB, _KOUT), jnp.int32)
    m0c = jnp.full((_RB, 1), jnp.inf, jnp.float32)
    i0c = jnp.full((_RB, 1), -1, jnp.int32)
    vals, idxs, _, _ = jax.lax.fori_loop(0, _KCAP, body,
                                         (vals0, idxs0, m0c, i0c))
    vals_ref[...] = vals
    idxs_ref[...] = idxs


def _rank_kernel(sid_ref, x_ref, xv_ref, rank_ref):
    x = x_ref[...]
    iota = jax.lax.broadcasted_iota(jnp.int32, x.shape, 1)
    sid = sid_ref[...]
    xv = jnp.max(jnp.where(iota == sid, x, _NEG_INF), axis=1, keepdims=True)
    rank_ref[...] = jnp.sum((x >= xv).astype(jnp.int32), axis=1, keepdims=True)
    xv_ref[...] = xv


def kernel(logits, temperature, top_p, token_lengths, output_lengths, top_k):
    logits = logits.astype(jnp.float32)
    B, V = logits.shape
    xpad = jnp.pad(logits, ((0, 0), (0, _VPAD - V)), constant_values=_NEG_INF)

    vals, idxs, lse2 = pl.pallas_call(
        _topk_lse_kernel,
        grid=(B // _RB,),
        in_specs=[pl.BlockSpec((_RB, _VPAD), lambda i: (i, 0))],
        out_specs=[
            pl.BlockSpec((_RB, _KOUT), lambda i: (i, 0)),
            pl.BlockSpec((_RB, _KOUT), lambda i: (i, 0)),
            pl.BlockSpec((_RB, 1), lambda i: (i, 0)),
        ],
        out_shape=[
            jax.ShapeDtypeStruct((B, _KOUT), jnp.float32),
            jax.ShapeDtypeStruct((B, _KOUT), jnp.int32),
            jax.ShapeDtypeStruct((B, 1), jnp.float32),
        ],
    )(xpad)
    lse = lse2[:, 0]

    v48 = vals[:, :_KCAP]
    i48 = idxs[:, :_KCAP]

    kidx = jnp.clip(jnp.asarray(top_k, jnp.int32) - 1, 0, _KCAP - 1)
    sv = v48 / temperature[:, None]
    thresh = jnp.take_along_axis(
        sv, jnp.broadcast_to(kidx.reshape(1, 1), (B, 1)), axis=-1)
    survivor = sv >= thresh
    q = jnp.where(survivor,
                  jax.nn.softmax(jnp.where(survivor, sv, _NEG_INF), axis=-1),
                  0.0)
    prefix = jnp.cumsum(q, axis=-1) - q
    pos = jnp.arange(_KCAP)[None, :]
    keep = survivor & ((prefix < top_p[:, None]) | (pos == 0))

    s_kept = jnp.sum(jnp.where(keep, jnp.exp(sv - sv[:, :1]), 0.0), axis=-1)
    max_logprobs = -jnp.log(s_kept)

    n_kept = jnp.sum(keep.astype(jnp.int32), axis=-1)
    order = jnp.argsort(jnp.where(keep, pos, _KCAP + pos), axis=-1)
    kept_ids_sorted = jnp.take_along_axis(i48, order[:, :10], axis=-1)

    small = jnp.arange(64)[None, :]
    kept_ids_all = jnp.where(keep, i48, -1)
    is_kept_id = jnp.any(small[:, :, None] == kept_ids_all[:, None, :], axis=-1)
    filler = jnp.sort(jnp.where(is_kept_id, 10000 + small, small), axis=-1)[:, :10]

    r10 = jnp.arange(10)[None, :]
    fill_pos = jnp.clip(r10 - n_kept[:, None], 0, 9)
    filler_pick = jnp.take_along_axis(filler, fill_pos, axis=-1)
    merged_ids = jnp.where(r10 < n_kept[:, None], kept_ids_sorted, filler_pick)

    lengths = token_lengths[jnp.clip(merged_ids, 0, token_lengths.shape[0] - 1)]
    longest = jnp.argmax(lengths, axis=-1)
    final = jnp.take_along_axis(merged_ids, longest[:, None], axis=-1)[:, 0]

    force = (output_lengths >= 64) & (max_logprobs < -0.7)
    sampled = jnp.where(force, 2, final)
    sampled = jnp.where(temperature < 1e-5, i48[:, 0], sampled).astype(jnp.int32)

    xv2, rank2 = pl.pallas_call(
        _rank_kernel,
        grid=(B // _RB,),
        in_specs=[
            pl.BlockSpec((_RB, 1), lambda i: (i, 0)),
            pl.BlockSpec((_RB, _VPAD), lambda i: (i, 0)),
        ],
        out_specs=[
            pl.BlockSpec((_RB, 1), lambda i: (i, 0)),
            pl.BlockSpec((_RB, 1), lambda i: (i, 0)),
        ],
        out_shape=[
            jax.ShapeDtypeStruct((B, 1), jnp.float32),
            jax.ShapeDtypeStruct((B, 1), jnp.int32),
        ],
    )(sampled[:, None], xpad)

    token_logprobs = xv2[:, 0] - lse
    indices = jnp.concatenate([sampled[:, None], i48[:, :20]], axis=1)
    lp = jnp.concatenate(
        [token_logprobs[:, None], v48[:, :20] - lse[:, None]], axis=1)
    return sampled[:, None], indices.astype(jnp.int32), lp, rank2[:, 0]
